# baseline (speedup 1.0000x reference)
import functools

import jax
import jax.numpy as jnp
from jax import lax
from jax.experimental import pallas as pl
from jax.experimental.pallas import tpu as pltpu
from jax.experimental.pallas import tpu_sc as plsc

_L = 16
_S = 4096
_ROW = 50
_M = 1000000
_BLK = 2048

_mesh = plsc.VectorSubcoreMesh(core_axis_name="c", subcore_axis_name="s")


def _log_f32(x):
    xi = plsc.bitcast(x, jnp.int32)
    e = (xi >> 23) - 127
    m = plsc.bitcast((xi & 0x7FFFFF) | (127 << 23), jnp.float32)
    t = (m - 1.0) / (m + 1.0)
    t2 = t * t
    poly = 1.0 + t2 * (1.0 / 3.0 + t2 * (1.0 / 5.0 + t2 * (1.0 / 7.0 + t2 / 9.0)))
    ln_m = 2.0 * t * poly
    return e.astype(jnp.float32) * 0.6931471805599453 + ln_m


@functools.partial(
    pl.kernel,
    out_type=jax.ShapeDtypeStruct((_L,), jnp.int32),
    mesh=_mesh,
    compiler_params=pltpu.CompilerParams(needs_layout_passes=False),
    cost_estimate=pl.CostEstimate(
        flops=10**8, transcendentals=10**6, bytes_accessed=10**8),
    scratch_types=[
        pltpu.VMEM((_L,), jnp.int32),
        pltpu.VMEM((_S,), jnp.float32),
        pltpu.VMEM((_L,), jnp.float32),
        pltpu.VMEM((_L,), jnp.int32),
        pltpu.VMEM((_L,), jnp.float32),
    ],
)
def _sc_scalars(combo_hbm, nlsd_hbm, len_ref, lp_ref, out_size_hbm,
                vc, vx, vlp, vwl, vwp):
    @pl.when(jnp.logical_and(lax.axis_index("c") == 0,
                             lax.axis_index("s") == 0))
    def _():
        pltpu.sync_copy(combo_hbm, vc)
        pltpu.sync_copy(nlsd_hbm, vx)

        lane = lax.iota(jnp.int32, _L)
        v = vc[...]
        zero = jnp.zeros((_L,), jnp.int32)
        idx = lax.reduce_max(jnp.where(lane < 8, v, zero), axes=(0,))
        vn = jnp.full((_L,), lax.reduce_max(jnp.where(lane >= 8, v, zero),
                                            axes=(0,)))

        base = pl.multiple_of(jnp.minimum((idx >> 3) << 3, _M - _L), 8)
        hit = lane == (idx - base)

        pltpu.sync_copy(len_ref.at[pl.ds(base, _L)], vwl)
        vwl[...] = jnp.where(hit, vn, vwl[...])
        pltpu.sync_copy(vwl, len_ref.at[pl.ds(base, _L)])

        def max_body(i, acc):
            return jnp.maximum(acc, vx[pl.ds(i * _L, _L)])

        mvec = lax.fori_loop(1, _S // _L, max_body, vx[pl.ds(0, _L)],
                             unroll=8)
        mmax = jnp.full((_L,), jnp.max(mvec))

        def sum_body(i, acc):
            return acc + jnp.exp(vx[pl.ds(i * _L, _L)] - mmax)

        svec = lax.fori_loop(0, _S // _L, sum_body,
                             jnp.zeros((_L,), jnp.float32), unroll=8)
        tot = jnp.full((_L,), jnp.sum(svec))
        vlp[...] = mmax + _log_f32(tot)

        pltpu.sync_copy(lp_ref.at[pl.ds(base, _L)], vwp)
        vwp[...] = jnp.where(hit, vlp[...], vwp[...])
        pltpu.sync_copy(vwp, lp_ref.at[pl.ds(base, _L)])

        vc[...] = v + 1
        pltpu.sync_copy(vc, out_size_hbm)


_NCHUNK = 13
_CHUNK = 601 * 128


def _copy_body(size_ref, nseq_ref, in_hbm, out_hbm, vwin, sem, semw):
    copies = []
    for k in range(_NCHUNK):
        off = k * _CHUNK
        sz = min(_CHUNK, _M - off)
        c = pltpu.make_async_copy(
            in_hbm.at[:, pl.ds(off, sz)], out_hbm.at[:, pl.ds(off, sz)], sem)
        c.start()
        copies.append(c)

    idx = size_ref[0]
    base128 = pl.multiple_of((idx >> 7) << 7, 128)
    win_in = pltpu.make_async_copy(
        in_hbm.at[:, pl.ds(base128, 128)], vwin, semw)
    win_in.start()
    win_in.wait()
    gl = base128 + lax.broadcasted_iota(jnp.int32, (_ROW, 128), 1)
    vwin[...] = jnp.where(gl == idx, nseq_ref[...], vwin[...])

    for c in copies:
        c.wait()
    win_out = pltpu.make_async_copy(
        vwin, out_hbm.at[:, pl.ds(base128, 128)], semw)
    win_out.start()
    win_out.wait()


def _copy_update(size1, nseq_b, seq_t):
    return pl.pallas_call(
        _copy_body,
        grid_spec=pltpu.PrefetchScalarGridSpec(
            num_scalar_prefetch=1,
            grid=(1,),
            in_specs=[
                pl.BlockSpec((_ROW, 128), lambda i, sref: (0, 0)),
                pl.BlockSpec(memory_space=pl.ANY),
            ],
            out_specs=pl.BlockSpec(memory_space=pl.ANY),
            scratch_shapes=[
                pltpu.VMEM((_ROW, 128), jnp.int32),
                pltpu.SemaphoreType.DMA,
                pltpu.SemaphoreType.DMA,
            ],
        ),
        out_shape=jax.ShapeDtypeStruct((_ROW, _M), jnp.int32),
    )(size1, nseq_b, seq_t)


def kernel(sequences, sequence_lengths, log_probabilities, size,
           node_sequence, node_sequence_length, node_log_state_distribution):
    size_i = jnp.asarray(size, jnp.int32)
    nsl_i = jnp.asarray(node_sequence_length, jnp.int32)
    combo = jnp.where(jnp.arange(_L) < 8, size_i, nsl_i)
    nseq_b = jnp.broadcast_to(
        jnp.asarray(node_sequence, jnp.int32)[:, None], (_ROW, 128))
    size1 = size_i.reshape(1)

    seq_t = sequences.T
    len_ref = jax.new_ref(sequence_lengths)
    lp_ref = jax.new_ref(log_probabilities)

    out16 = _sc_scalars(combo, node_log_state_distribution, len_ref, lp_ref)
    seq_new_t = _copy_update(size1, nseq_b, seq_t)

    return seq_new_t.T, len_ref[...], lp_ref[...], out16[0]


# speedup vs baseline: 37.6560x; 37.6560x over previous
import functools

import jax
import jax.numpy as jnp
from jax import lax
from jax.experimental import pallas as pl
from jax.experimental.pallas import tpu as pltpu
from jax.experimental.pallas import tpu_sc as plsc

_L = 16
_S = 4096
_ROW = 50
_M = 1000000

_mesh = plsc.VectorSubcoreMesh(core_axis_name="c", subcore_axis_name="s")


def _log_f32(x):
    xi = plsc.bitcast(x, jnp.int32)
    e = (xi >> 23) - 127
    m = plsc.bitcast((xi & 0x7FFFFF) | (127 << 23), jnp.float32)
    t = (m - 1.0) / (m + 1.0)
    t2 = t * t
    poly = 1.0 + t2 * (1.0 / 3.0 + t2 * (1.0 / 5.0 + t2 * (1.0 / 7.0 + t2 / 9.0)))
    ln_m = 2.0 * t * poly
    return e.astype(jnp.float32) * 0.6931471805599453 + ln_m


@functools.partial(
    pl.kernel,
    out_type=jax.ShapeDtypeStruct((_L,), jnp.int32),
    mesh=_mesh,
    compiler_params=pltpu.CompilerParams(needs_layout_passes=False),
    cost_estimate=pl.CostEstimate(
        flops=10**8, transcendentals=10**6, bytes_accessed=10**8),
    scratch_types=[
        pltpu.VMEM((_L,), jnp.int32),
        pltpu.VMEM((_S,), jnp.float32),
        pltpu.VMEM((_L,), jnp.float32),
        pltpu.VMEM((_L,), jnp.int32),
        pltpu.VMEM((_L,), jnp.float32),
    ],
)
def _sc_scalars(combo_hbm, nlsd_hbm, len_ref, lp_ref, out_size_hbm,
                vc, vx, vlp, vwl, vwp):
    @pl.when(jnp.logical_and(lax.axis_index("c") == 0,
                             lax.axis_index("s") == 0))
    def _():
        pltpu.sync_copy(combo_hbm, vc)
        pltpu.sync_copy(nlsd_hbm, vx)

        lane = lax.iota(jnp.int32, _L)
        v = vc[...]
        zero = jnp.zeros((_L,), jnp.int32)
        idx = lax.reduce_max(jnp.where(lane < 8, v, zero), axes=(0,))
        vn = jnp.full((_L,), lax.reduce_max(jnp.where(lane >= 8, v, zero),
                                            axes=(0,)))

        base = pl.multiple_of(jnp.minimum((idx >> 3) << 3, _M - _L), 8)
        hit = lane == (idx - base)

        pltpu.sync_copy(len_ref.at[pl.ds(base, _L)], vwl)
        vwl[...] = jnp.where(hit, vn, vwl[...])
        pltpu.sync_copy(vwl, len_ref.at[pl.ds(base, _L)])

        def max_body(i, acc):
            return jnp.maximum(acc, vx[pl.ds(i * _L, _L)])

        mvec = lax.fori_loop(1, _S // _L, max_body, vx[pl.ds(0, _L)],
                             unroll=8)
        mmax = jnp.full((_L,), jnp.max(mvec))

        def sum_body(i, acc):
            return acc + jnp.exp(vx[pl.ds(i * _L, _L)] - mmax)

        svec = lax.fori_loop(0, _S // _L, sum_body,
                             jnp.zeros((_L,), jnp.float32), unroll=8)
        tot = jnp.full((_L,), jnp.sum(svec))
        vlp[...] = mmax + _log_f32(tot)

        pltpu.sync_copy(lp_ref.at[pl.ds(base, _L)], vwp)
        vwp[...] = jnp.where(hit, vlp[...], vwp[...])
        pltpu.sync_copy(vwp, lp_ref.at[pl.ds(base, _L)])

        vc[...] = v + 1
        pltpu.sync_copy(vc, out_size_hbm)


_CH = 7936
_NBUF = 4
_NCH = 126
_TAIL = _NCH * _CH


def _copy_body(size_ref, nseq_ref, in_hbm, out_hbm,
               b0, b1, b2, b3, vwin, vtail,
               si0, si1, si2, si3, so0, so1, so2, so3, semw):
    bufs = (b0, b1, b2, b3)
    sin = (si0, si1, si2, si3)
    sout = (so0, so1, so2, so3)

    ins = [None] * _NCH
    outs = [None] * _NCH
    for k in range(_NCH + 1):
        if k < _NCH:
            b = k % _NBUF
            if k >= _NBUF:
                outs[k - _NBUF].wait()
            off = k * _CH
            ins[k] = pltpu.make_async_copy(
                in_hbm.at[:, pl.ds(off, _CH)], bufs[b], sin[b])
            ins[k].start()
        if k >= 1:
            j = k - 1
            b = j % _NBUF
            ins[j].wait()
            outs[j] = pltpu.make_async_copy(
                bufs[b], out_hbm.at[:, pl.ds(j * _CH, _CH)], sout[b])
            outs[j].start()

    idx = size_ref[0]
    toff = pl.multiple_of((idx >> 30) + _TAIL, 128)
    tail_in = pltpu.make_async_copy(
        in_hbm.at[:, pl.ds(toff, 128)], vtail, semw)
    tail_in.start()

    base128 = pl.multiple_of((idx >> 7) << 7, 128)
    win_in = pltpu.make_async_copy(
        in_hbm.at[:, pl.ds(base128, 128)], vwin, semw)
    tail_in.wait()
    win_in.start()
    win_in.wait()
    gl = base128 + lax.broadcasted_iota(jnp.int32, (_ROW, 128), 1)
    col = jnp.broadcast_to(
        jnp.transpose(nseq_ref[...], (1, 0)), (_ROW, 128))
    vwin[...] = jnp.where(gl == idx, col, vwin[...])

    for j in range(max(0, _NCH - _NBUF), _NCH):
        outs[j].wait()
    tail_out = pltpu.make_async_copy(
        vtail, out_hbm.at[:, pl.ds(toff, 128)], semw)
    tail_out.start()
    tail_out.wait()
    win_out = pltpu.make_async_copy(
        vwin, out_hbm.at[:, pl.ds(base128, 128)], semw)
    win_out.start()
    win_out.wait()


def _paste(size1, nseq2d, seq_t):
    return pl.pallas_call(
        _copy_body,
        grid_spec=pltpu.PrefetchScalarGridSpec(
            num_scalar_prefetch=1,
            grid=(1,),
            in_specs=[
                pl.BlockSpec((1, _ROW), lambda i, sref: (0, 0)),
                pl.BlockSpec(memory_space=pl.ANY),
            ],
            out_specs=pl.BlockSpec(memory_space=pl.ANY),
            scratch_shapes=(
                [pltpu.VMEM((_ROW, _CH), jnp.int32) for _ in range(_NBUF)]
                + [pltpu.VMEM((_ROW, 128), jnp.int32)] * 2
                + [pltpu.SemaphoreType.DMA] * (2 * _NBUF + 1)
            ),
        ),
        out_shape=jax.ShapeDtypeStruct((_ROW, _M), jnp.int32),
    )(size1, nseq2d, seq_t)


def kernel(sequences, sequence_lengths, log_probabilities, size,
           node_sequence, node_sequence_length, node_log_state_distribution):
    size_i = jnp.asarray(size, jnp.int32)
    nsl_i = jnp.asarray(node_sequence_length, jnp.int32)
    combo = jnp.where(jnp.arange(_L) < 8, size_i, nsl_i)
    nseq2d = jnp.asarray(node_sequence, jnp.int32).reshape(1, _ROW)
    size1 = size_i.reshape(1)

    seq_t = sequences.T
    len_ref = jax.new_ref(sequence_lengths)
    lp_ref = jax.new_ref(log_probabilities)

    out16 = _sc_scalars(combo, node_log_state_distribution, len_ref, lp_ref)
    seq_new_t = _paste(size1, nseq2d, seq_t)

    return seq_new_t.T, len_ref[...], lp_ref[...], out16[0]
